# baseline (device time: 16894 ns/iter reference)
import jax
import jax.numpy as jnp
from jax import lax
from jax.experimental import pallas as pl
from jax.experimental.pallas import tpu as pltpu

N_PARTNERS = 7


def kernel(x):
    m, n = x.shape[-2], x.shape[-1]
    h = m // 2
    x2 = x.reshape(m, n)

    def body(
        x_ref,
        out_ref,
        acc0,
        acc1,
        acc2,
        recv_x,
        recv_a1,
        recv_b1,
        recv_a2,
        recv_b2,
        sem_x,
        send1,
        send2,
        rsem1,
        rsem2,
    ):
        my_x = lax.axis_index("x")
        my_y = lax.axis_index("y")
        my_z = lax.axis_index("z")
        me = (my_x, my_y, my_z)

        def y_at(off):
            return (my_x, jnp.bitwise_and(my_y + off, 3), my_z)

        def z_at(off):
            return (my_x, my_y, jnp.bitwise_and(my_z + off, 3))

        partners = [(1 - my_x, my_y, my_z)]
        partners += [y_at(j) for j in (1, 2, 3)]
        partners += [z_at(j) for j in (1, 2, 3)]

        barrier_sem = pltpu.get_barrier_semaphore()
        for p in partners:
            pl.semaphore_signal(
                barrier_sem, inc=1,
                device_id=p, device_id_type=pl.DeviceIdType.MESH,
            )
        acc0[...] = x_ref[...].astype(jnp.bfloat16)
        pl.semaphore_wait(barrier_sem, N_PARTNERS)

        a = pl.ds(0, h)
        b = pl.ds(h, h)

        def bcast(src, dst_slots, send_sems, recv_sems, at):
            rdmas = []
            for j in (1, 2, 3):
                r = pltpu.make_async_remote_copy(
                    src_ref=src,
                    dst_ref=dst_slots.at[4 - j],
                    send_sem=send_sems.at[j - 1],
                    recv_sem=recv_sems.at[4 - j],
                    device_id=at(j),
                    device_id_type=pl.DeviceIdType.MESH,
                )
                r.start()
                rdmas.append(r)
            return rdmas

        def wait_recvs(slots, recv_sems):
            for s in (1, 2, 3):
                r = pltpu.make_async_remote_copy(
                    src_ref=slots.at[s],
                    dst_ref=slots.at[s],
                    send_sem=sem_x.at[0, 0],
                    recv_sem=recv_sems.at[s],
                    device_id=me,
                    device_id_type=pl.DeviceIdType.MESH,
                )
                r.wait_recv()

        xch_a = pltpu.make_async_remote_copy(
            src_ref=acc0.at[a],
            dst_ref=recv_x.at[a],
            send_sem=sem_x.at[0, 0],
            recv_sem=sem_x.at[0, 1],
            device_id=partners[0],
            device_id_type=pl.DeviceIdType.MESH,
        )
        xch_a.start()
        xch_b = pltpu.make_async_remote_copy(
            src_ref=acc0.at[b],
            dst_ref=recv_x.at[b],
            send_sem=sem_x.at[1, 0],
            recv_sem=sem_x.at[1, 1],
            device_id=partners[0],
            device_id_type=pl.DeviceIdType.MESH,
        )
        xch_b.start()

        xch_a.wait_recv()
        acc1[a, :] = acc0[a, :] + recv_x[a, :]
        s1a = bcast(acc1.at[a], recv_a1, send1.at[0], rsem1.at[0], y_at)

        xch_b.wait_recv()
        acc1[b, :] = acc0[b, :] + recv_x[b, :]
        s1b = bcast(acc1.at[b], recv_b1, send1.at[1], rsem1.at[1], z_at)

        wait_recvs(recv_a1, rsem1.at[0])
        acc2[a, :] = acc1[a, :] + recv_a1[1] + recv_a1[2] + recv_a1[3]
        s2a = bcast(acc2.at[a], recv_a2, send2.at[0], rsem2.at[0], z_at)

        wait_recvs(recv_b1, rsem1.at[1])
        acc2[b, :] = acc1[b, :] + recv_b1[1] + recv_b1[2] + recv_b1[3]
        s2b = bcast(acc2.at[b], recv_b2, send2.at[1], rsem2.at[1], y_at)

        wait_recvs(recv_a2, rsem2.at[0])
        out_ref[a, :] = (
            acc2[a, :] + recv_a2[1] + recv_a2[2] + recv_a2[3]
        ).astype(jnp.float32)

        wait_recvs(recv_b2, rsem2.at[1])
        out_ref[b, :] = (
            acc2[b, :] + recv_b2[1] + recv_b2[2] + recv_b2[3]
        ).astype(jnp.float32)

        xch_a.wait_send()
        xch_b.wait_send()
        for r in s1a + s1b + s2a + s2b:
            r.wait_send()

    return pl.pallas_call(
        body,
        out_shape=jax.ShapeDtypeStruct((m, n), jnp.float32),
        in_specs=[pl.BlockSpec(memory_space=pltpu.VMEM)],
        out_specs=pl.BlockSpec(memory_space=pltpu.VMEM),
        scratch_shapes=[
            pltpu.VMEM((m, n), jnp.bfloat16),
            pltpu.VMEM((m, n), jnp.bfloat16),
            pltpu.VMEM((m, n), jnp.bfloat16),
            pltpu.VMEM((m, n), jnp.bfloat16),
            pltpu.VMEM((4, h, n), jnp.bfloat16),
            pltpu.VMEM((4, h, n), jnp.bfloat16),
            pltpu.VMEM((4, h, n), jnp.bfloat16),
            pltpu.VMEM((4, h, n), jnp.bfloat16),
            pltpu.SemaphoreType.DMA((2, 2)),
            pltpu.SemaphoreType.DMA((2, 3)),
            pltpu.SemaphoreType.DMA((2, 3)),
            pltpu.SemaphoreType.DMA((2, 4)),
            pltpu.SemaphoreType.DMA((2, 4)),
        ],
        compiler_params=pltpu.CompilerParams(collective_id=0),
    )(x2)
